# Initial kernel scaffold; baseline (speedup 1.0000x reference)
import jax
import jax.numpy as jnp
from jax.experimental import pallas as pl


def kernel(activation, ema_activation, pseudo_label, queue_list):
    raise NotImplementedError("write your pallas kernel here")


# speedup vs baseline: 3.2517x; 3.2517x over previous
import jax
import jax.numpy as jnp
from jax.experimental import pallas as pl
from jax.experimental.pallas import tpu as pltpu

PROJ_DIM = 128
CLASS_NUM = 7
QUEUE_SIZE = 150
NQ = CLASS_NUM * QUEUE_SIZE
TEMPERATURE = 0.5
BLK = 1024


def _pgc_kernel(act_ref, ema_ref, lab_ref, q_ref, out_ref):
    a = act_ref[...]
    e = ema_ref[...]
    p = lab_ref[...]

    an = a * (1.0 / jnp.maximum(jnp.sqrt(jnp.sum(a * a, axis=1, keepdims=True)), 1e-12))
    en = e * (1.0 / jnp.maximum(jnp.sqrt(jnp.sum(e * e, axis=1, keepdims=True)), 1e-12))

    l_pos = jnp.exp(jnp.sum(an * en, axis=1, keepdims=True) / TEMPERATURE)

    rowmax = jnp.max(p, axis=1, keepdims=True)
    cls_idx = jax.lax.broadcasted_iota(jnp.int32, p.shape, 1)
    label = jnp.min(jnp.where(p == rowmax, cls_idx, CLASS_NUM), axis=1, keepdims=True)

    q = q_ref[...]
    s = jax.lax.dot_general(
        an, q, (((1,), (1,)), ((), ())), preferred_element_type=jnp.float32
    )
    sims = jnp.exp(s / TEMPERATURE)
    total = jnp.sum(sims, axis=1, keepdims=True)
    denom = l_pos + total

    col = jax.lax.broadcasted_iota(jnp.int32, sims.shape, 1)
    lo = label * QUEUE_SIZE
    mask = (col >= lo) & (col < lo + QUEUE_SIZE)

    val = -jnp.log(sims / denom + 1e-8)
    pos_sum = jnp.sum(jnp.where(mask, val, 0.0), axis=1, keepdims=True)
    per = (-jnp.log(l_pos / denom + 1e-8) + pos_sum) / (QUEUE_SIZE + 1)
    partial = jnp.sum(per)

    @pl.when(pl.program_id(0) == 0)
    def _init():
        out_ref[0, 0] = 0.0

    out_ref[0, 0] += partial


def kernel(activation, ema_activation, pseudo_label, queue_list):
    batch = activation.shape[0]
    grid = batch // BLK
    out = pl.pallas_call(
        _pgc_kernel,
        grid=(grid,),
        in_specs=[
            pl.BlockSpec((BLK, PROJ_DIM), lambda i: (i, 0)),
            pl.BlockSpec((BLK, PROJ_DIM), lambda i: (i, 0)),
            pl.BlockSpec((BLK, CLASS_NUM), lambda i: (i, 0)),
            pl.BlockSpec((NQ, PROJ_DIM), lambda i: (0, 0)),
        ],
        out_specs=pl.BlockSpec((1, 1), lambda i: (0, 0), memory_space=pltpu.SMEM),
        out_shape=jax.ShapeDtypeStruct((1, 1), jnp.float32),
    )(activation, ema_activation, pseudo_label, queue_list)
    return out[0, 0] / batch


# speedup vs baseline: 3.3604x; 1.0334x over previous
import jax
import jax.numpy as jnp
import numpy as np
from jax.experimental import pallas as pl
from jax.experimental.pallas import tpu as pltpu

PROJ_DIM = 128
CLASS_NUM = 7
QUEUE_SIZE = 150
NQ = CLASS_NUM * QUEUE_SIZE
TEMPERATURE = 0.5
INV_T = 1.0 / TEMPERATURE
BLK = 1024
CPAD = 8


def _pgc_kernel(act_ref, ema_ref, lab_ref, q_ref, seg_ref, out_ref):
    a = act_ref[...]
    e = ema_ref[...]
    p = lab_ref[...]

    inv_a = 1.0 / jnp.maximum(jnp.sqrt(jnp.sum(a * a, axis=1, keepdims=True)), 1e-12)
    inv_e = 1.0 / jnp.maximum(jnp.sqrt(jnp.sum(e * e, axis=1, keepdims=True)), 1e-12)

    spos = jnp.sum(a * e, axis=1, keepdims=True) * (inv_a * inv_e * INV_T)

    an_t = a * (inv_a * INV_T)
    s = jax.lax.dot_general(
        an_t, q_ref[...], (((1,), (1,)), ((), ())), preferred_element_type=jnp.float32
    )
    sims = jnp.exp(s)
    rcp = 1.0 / sims

    seg = seg_ref[...]
    cls_exp = jax.lax.dot_general(
        sims, seg, (((1,), (0,)), ((), ())), preferred_element_type=jnp.float32
    )
    cls_s = jax.lax.dot_general(
        s, seg, (((1,), (0,)), ((), ())), preferred_element_type=jnp.float32
    )
    cls_rcp = jax.lax.dot_general(
        rcp, seg, (((1,), (0,)), ((), ())), preferred_element_type=jnp.float32
    )

    rowmax = jnp.max(p, axis=1, keepdims=True)
    cls_idx = jax.lax.broadcasted_iota(jnp.int32, p.shape, 1)
    label = jnp.min(jnp.where(p == rowmax, cls_idx, CLASS_NUM), axis=1, keepdims=True)
    onehot = (jax.lax.broadcasted_iota(jnp.int32, cls_exp.shape, 1) == label).astype(
        jnp.float32
    )

    total = jnp.sum(cls_exp, axis=1, keepdims=True)
    denom = jnp.exp(spos) + total
    log_denom = jnp.log(denom)

    pos_s = jnp.sum(onehot * cls_s, axis=1, keepdims=True)
    pos_rcp = jnp.sum(onehot * cls_rcp, axis=1, keepdims=True)

    per = (
        (QUEUE_SIZE + 1) * log_denom
        - spos
        - pos_s
        - 1e-8 * denom * (jnp.exp(-spos) + pos_rcp)
    ) * (1.0 / (QUEUE_SIZE + 1))
    partial = jnp.sum(per)

    @pl.when(pl.program_id(0) == 0)
    def _init():
        out_ref[0, 0] = 0.0

    out_ref[0, 0] += partial


def kernel(activation, ema_activation, pseudo_label, queue_list):
    batch = activation.shape[0]
    grid = batch // BLK
    seg_np = np.zeros((NQ, CPAD), dtype=np.float32)
    for c in range(CLASS_NUM):
        seg_np[c * QUEUE_SIZE : (c + 1) * QUEUE_SIZE, c] = 1.0
    seg = jnp.asarray(seg_np)
    out = pl.pallas_call(
        _pgc_kernel,
        grid=(grid,),
        in_specs=[
            pl.BlockSpec((BLK, PROJ_DIM), lambda i: (i, 0)),
            pl.BlockSpec((BLK, PROJ_DIM), lambda i: (i, 0)),
            pl.BlockSpec((BLK, CLASS_NUM), lambda i: (i, 0)),
            pl.BlockSpec((NQ, PROJ_DIM), lambda i: (0, 0)),
            pl.BlockSpec((NQ, CPAD), lambda i: (0, 0)),
        ],
        out_specs=pl.BlockSpec((1, 1), lambda i: (0, 0), memory_space=pltpu.SMEM),
        out_shape=jax.ShapeDtypeStruct((1, 1), jnp.float32),
    )(activation, ema_activation, pseudo_label, queue_list, seg)
    return out[0, 0] / batch


# speedup vs baseline: 4.6646x; 1.3881x over previous
import jax
import jax.numpy as jnp
import numpy as np
from jax.experimental import pallas as pl
from jax.experimental.pallas import tpu as pltpu

PROJ_DIM = 128
CLASS_NUM = 7
QUEUE_SIZE = 150
NQ = CLASS_NUM * QUEUE_SIZE
TEMPERATURE = 0.5
INV_T = 1.0 / TEMPERATURE
BLK = 1024
CPAD = 8


def _pgc_kernel(act_ref, ema_ref, lab_ref, q_ref, seg_ref, out_ref):
    a = act_ref[...]
    e = ema_ref[...]
    p = lab_ref[...]
    q = q_ref[...]

    inv_a = 1.0 / jnp.maximum(jnp.sqrt(jnp.sum(a * a, axis=1, keepdims=True)), 1e-12)
    inv_e = 1.0 / jnp.maximum(jnp.sqrt(jnp.sum(e * e, axis=1, keepdims=True)), 1e-12)

    spos = jnp.sum(a * e, axis=1, keepdims=True) * (inv_a * inv_e * INV_T)

    an_t = a * (inv_a * INV_T)
    s = jax.lax.dot_general(
        an_t, q, (((1,), (1,)), ((), ())), preferred_element_type=jnp.float32
    )
    sims = jnp.exp(s)
    total = jnp.sum(sims, axis=1, keepdims=True)

    qsum = jax.lax.dot_general(
        seg_ref[...], q, (((0,), (0,)), ((), ())), preferred_element_type=jnp.float32
    )
    cls_s = jax.lax.dot_general(
        an_t, qsum, (((1,), (1,)), ((), ())), preferred_element_type=jnp.float32
    )

    rowmax = jnp.max(p, axis=1, keepdims=True)
    cls_idx = jax.lax.broadcasted_iota(jnp.int32, p.shape, 1)
    label = jnp.min(jnp.where(p == rowmax, cls_idx, CLASS_NUM), axis=1, keepdims=True)
    onehot = (jax.lax.broadcasted_iota(jnp.int32, cls_s.shape, 1) == label).astype(
        jnp.float32
    )
    pos_s = jnp.sum(onehot * cls_s, axis=1, keepdims=True)

    denom = jnp.exp(spos) + total
    per = (QUEUE_SIZE + 1) * jnp.log(denom) - spos - pos_s
    partial = jnp.sum(per) * (1.0 / (QUEUE_SIZE + 1))

    @pl.when(pl.program_id(0) == 0)
    def _init():
        out_ref[0, 0] = 0.0

    out_ref[0, 0] += partial


def kernel(activation, ema_activation, pseudo_label, queue_list):
    batch = activation.shape[0]
    grid = batch // BLK
    seg_np = np.zeros((NQ, CPAD), dtype=np.float32)
    for c in range(CLASS_NUM):
        seg_np[c * QUEUE_SIZE : (c + 1) * QUEUE_SIZE, c] = 1.0
    seg = jnp.asarray(seg_np)
    out = pl.pallas_call(
        _pgc_kernel,
        grid=(grid,),
        in_specs=[
            pl.BlockSpec((BLK, PROJ_DIM), lambda i: (i, 0)),
            pl.BlockSpec((BLK, PROJ_DIM), lambda i: (i, 0)),
            pl.BlockSpec((BLK, CLASS_NUM), lambda i: (i, 0)),
            pl.BlockSpec((NQ, PROJ_DIM), lambda i: (0, 0)),
            pl.BlockSpec((NQ, CPAD), lambda i: (0, 0)),
        ],
        out_specs=pl.BlockSpec((1, 1), lambda i: (0, 0), memory_space=pltpu.SMEM),
        out_shape=jax.ShapeDtypeStruct((1, 1), jnp.float32),
    )(activation, ema_activation, pseudo_label, queue_list, seg)
    return out[0, 0] / batch


# speedup vs baseline: 4.9925x; 1.0703x over previous
import jax
import jax.numpy as jnp
import numpy as np
from jax.experimental import pallas as pl
from jax.experimental.pallas import tpu as pltpu

PROJ_DIM = 128
CLASS_NUM = 7
QUEUE_SIZE = 150
NQ = CLASS_NUM * QUEUE_SIZE
TEMPERATURE = 0.5
INV_T = 1.0 / TEMPERATURE
BLK = 2048
CPAD = 8


def _rowsum(x):
    ones = jnp.ones((x.shape[1], 1), dtype=jnp.float32)
    return jax.lax.dot_general(
        x, ones, (((1,), (0,)), ((), ())), preferred_element_type=jnp.float32
    )


def _pgc_kernel(act_ref, ema_ref, lab_ref, q_ref, seg_ref, out_ref):
    a = act_ref[...]
    e = ema_ref[...]
    p = lab_ref[...]
    q = q_ref[...]

    inv_a = 1.0 / jnp.maximum(jnp.sqrt(_rowsum(a * a)), 1e-12)
    inv_e = 1.0 / jnp.maximum(jnp.sqrt(_rowsum(e * e)), 1e-12)

    spos = _rowsum(a * e) * (inv_a * inv_e * INV_T)

    an_t = a * (inv_a * INV_T)
    s = jax.lax.dot_general(
        an_t, q, (((1,), (1,)), ((), ())), preferred_element_type=jnp.float32
    )
    sims = jnp.exp(s)
    total = _rowsum(sims)

    qsum = jax.lax.dot_general(
        seg_ref[...], q, (((0,), (0,)), ((), ())), preferred_element_type=jnp.float32
    )
    cls_s = jax.lax.dot_general(
        an_t, qsum, (((1,), (1,)), ((), ())), preferred_element_type=jnp.float32
    )

    rowmax = jnp.max(p, axis=1, keepdims=True)
    cls_idx = jax.lax.broadcasted_iota(jnp.int32, p.shape, 1)
    label = jnp.min(jnp.where(p == rowmax, cls_idx, CLASS_NUM), axis=1, keepdims=True)
    onehot = (jax.lax.broadcasted_iota(jnp.int32, cls_s.shape, 1) == label).astype(
        jnp.float32
    )
    pos_s = jnp.sum(onehot * cls_s, axis=1, keepdims=True)

    denom = jnp.exp(spos) + total
    per = (QUEUE_SIZE + 1) * jnp.log(denom) - spos - pos_s
    partial = jnp.sum(per) * (1.0 / (QUEUE_SIZE + 1))

    @pl.when(pl.program_id(0) == 0)
    def _init():
        out_ref[0, 0] = 0.0

    out_ref[0, 0] += partial


def kernel(activation, ema_activation, pseudo_label, queue_list):
    batch = activation.shape[0]
    grid = batch // BLK
    seg_np = np.zeros((NQ, CPAD), dtype=np.float32)
    for c in range(CLASS_NUM):
        seg_np[c * QUEUE_SIZE : (c + 1) * QUEUE_SIZE, c] = 1.0
    seg = jnp.asarray(seg_np)
    out = pl.pallas_call(
        _pgc_kernel,
        grid=(grid,),
        in_specs=[
            pl.BlockSpec((BLK, PROJ_DIM), lambda i: (i, 0)),
            pl.BlockSpec((BLK, PROJ_DIM), lambda i: (i, 0)),
            pl.BlockSpec((BLK, CLASS_NUM), lambda i: (i, 0)),
            pl.BlockSpec((NQ, PROJ_DIM), lambda i: (0, 0)),
            pl.BlockSpec((NQ, CPAD), lambda i: (0, 0)),
        ],
        out_specs=pl.BlockSpec((1, 1), lambda i: (0, 0), memory_space=pltpu.SMEM),
        out_shape=jax.ShapeDtypeStruct((1, 1), jnp.float32),
    )(activation, ema_activation, pseudo_label, queue_list, seg)
    return out[0, 0] / batch


# speedup vs baseline: 5.5420x; 1.1101x over previous
import jax
import jax.numpy as jnp
import numpy as np
from jax.experimental import pallas as pl
from jax.experimental.pallas import tpu as pltpu

PROJ_DIM = 128
CLASS_NUM = 7
QUEUE_SIZE = 150
NQ = CLASS_NUM * QUEUE_SIZE
TEMPERATURE = 0.5
INV_T = 1.0 / TEMPERATURE
T_SQ = TEMPERATURE * TEMPERATURE
BLK = 2048
CPAD = 8


def _rowsum_mxu(x):
    ones = jnp.ones((x.shape[1], 1), dtype=jnp.float32)
    return jax.lax.dot_general(
        x, ones, (((1,), (0,)), ((), ())), preferred_element_type=jnp.float32
    )


def _pgc_kernel(act_ref, ema_ref, lab_ref, q_ref, seg_ref, out_ref):
    a = act_ref[...]
    e = ema_ref[...]
    p = lab_ref[...]
    q = q_ref[...]

    ssa = _rowsum_mxu(a * a)
    sse = _rowsum_mxu(e * e)
    dot_ae = _rowsum_mxu(a * e)

    inv_at = jax.lax.rsqrt(jnp.maximum(ssa * T_SQ, 1e-24 * T_SQ))
    inv_e = jax.lax.rsqrt(jnp.maximum(sse, 1e-24))
    spos = dot_ae * inv_at * inv_e

    an_t = a * inv_at
    s = jax.lax.dot_general(
        an_t, q, (((1,), (1,)), ((), ())), preferred_element_type=jnp.float32
    )
    sims = jnp.exp(s)
    total = jnp.sum(sims, axis=1, keepdims=True)

    rowmax = jnp.max(p, axis=1, keepdims=True)
    cls_idx = jax.lax.broadcasted_iota(jnp.int32, p.shape, 1)
    label = jnp.min(jnp.where(p == rowmax, cls_idx, CLASS_NUM), axis=1, keepdims=True)
    onehot = (
        jax.lax.broadcasted_iota(jnp.int32, (p.shape[0], CPAD), 1) == label
    ).astype(jnp.float32)

    qsum = jax.lax.dot_general(
        seg_ref[...], q, (((0,), (0,)), ((), ())), preferred_element_type=jnp.float32
    )
    g = jax.lax.dot_general(
        onehot, an_t, (((0,), (0,)), ((), ())), preferred_element_type=jnp.float32
    )
    pos_total = jnp.sum(qsum * g)

    denom = jnp.exp(spos) + total
    partial = (
        (QUEUE_SIZE + 1) * jnp.sum(jnp.log(denom)) - jnp.sum(spos) - pos_total
    ) * (1.0 / (QUEUE_SIZE + 1))

    @pl.when(pl.program_id(0) == 0)
    def _init():
        out_ref[0, 0] = 0.0

    out_ref[0, 0] += partial


def kernel(activation, ema_activation, pseudo_label, queue_list):
    batch = activation.shape[0]
    grid = batch // BLK
    seg_np = np.zeros((NQ, CPAD), dtype=np.float32)
    for c in range(CLASS_NUM):
        seg_np[c * QUEUE_SIZE : (c + 1) * QUEUE_SIZE, c] = 1.0
    seg = jnp.asarray(seg_np)
    out = pl.pallas_call(
        _pgc_kernel,
        grid=(grid,),
        in_specs=[
            pl.BlockSpec((BLK, PROJ_DIM), lambda i: (i, 0)),
            pl.BlockSpec((BLK, PROJ_DIM), lambda i: (i, 0)),
            pl.BlockSpec((BLK, CLASS_NUM), lambda i: (i, 0)),
            pl.BlockSpec((NQ, PROJ_DIM), lambda i: (0, 0)),
            pl.BlockSpec((NQ, CPAD), lambda i: (0, 0)),
        ],
        out_specs=pl.BlockSpec((1, 1), lambda i: (0, 0), memory_space=pltpu.SMEM),
        out_shape=jax.ShapeDtypeStruct((1, 1), jnp.float32),
    )(activation, ema_activation, pseudo_label, queue_list, seg)
    return out[0, 0] / batch


# speedup vs baseline: 8.8106x; 1.5898x over previous
import jax
import jax.numpy as jnp
import numpy as np
from jax.experimental import pallas as pl
from jax.experimental.pallas import tpu as pltpu

PROJ_DIM = 128
CLASS_NUM = 7
QUEUE_SIZE = 150
NQ = CLASS_NUM * QUEUE_SIZE
TEMPERATURE = 0.5
INV_T = 1.0 / TEMPERATURE
T_SQ = TEMPERATURE * TEMPERATURE
BLK = 2048
CPAD = 8


def _rowsum_mxu(x):
    ones = jnp.ones((x.shape[1], 1), dtype=jnp.float32)
    return jax.lax.dot_general(
        x, ones, (((1,), (0,)), ((), ())), preferred_element_type=jnp.float32
    )


def _pgc_kernel(act_ref, ema_ref, lab_ref, q_ref, seg_ref, out_ref):
    a = act_ref[...]
    e = ema_ref[...]
    pt = lab_ref[...]
    q = q_ref[...]

    ssa = _rowsum_mxu(a * a)
    sse = _rowsum_mxu(e * e)
    dot_ae = _rowsum_mxu(a * e)

    inv_at = jax.lax.rsqrt(jnp.maximum(ssa * T_SQ, 1e-24 * T_SQ))
    inv_e = jax.lax.rsqrt(jnp.maximum(sse, 1e-24))
    spos = dot_ae * inv_at * inv_e

    an_t = a * inv_at
    s = jax.lax.dot_general(
        an_t, q, (((1,), (1,)), ((), ())), preferred_element_type=jnp.float32
    )
    sims = jnp.exp(s)
    total = jnp.sum(sims, axis=1, keepdims=True)

    colmax = jnp.max(pt, axis=0, keepdims=True)
    cls_idx = jax.lax.broadcasted_iota(jnp.int32, pt.shape, 0)
    label = jnp.min(
        jnp.where(pt == colmax, cls_idx, CLASS_NUM), axis=0, keepdims=True
    )
    onehot_t = (
        jax.lax.broadcasted_iota(jnp.int32, (CPAD, pt.shape[1]), 0) == label
    ).astype(jnp.float32)

    qsum = jax.lax.dot_general(
        seg_ref[...], q, (((0,), (0,)), ((), ())), preferred_element_type=jnp.float32
    )
    g = jax.lax.dot_general(
        onehot_t, an_t, (((1,), (0,)), ((), ())), preferred_element_type=jnp.float32
    )
    pos_total = jnp.sum(qsum * g)

    denom = jnp.exp(spos) + total
    partial = (
        (QUEUE_SIZE + 1) * jnp.sum(jnp.log(denom)) - jnp.sum(spos) - pos_total
    ) * (1.0 / (QUEUE_SIZE + 1))

    @pl.when(pl.program_id(0) == 0)
    def _init():
        out_ref[0, 0] = 0.0

    out_ref[0, 0] += partial


def kernel(activation, ema_activation, pseudo_label, queue_list):
    batch = activation.shape[0]
    grid = batch // BLK
    seg_np = np.zeros((NQ, CPAD), dtype=np.float32)
    for c in range(CLASS_NUM):
        seg_np[c * QUEUE_SIZE : (c + 1) * QUEUE_SIZE, c] = 1.0
    seg = jnp.asarray(seg_np)
    pl_t = pseudo_label.T
    out = pl.pallas_call(
        _pgc_kernel,
        grid=(grid,),
        in_specs=[
            pl.BlockSpec((BLK, PROJ_DIM), lambda i: (i, 0)),
            pl.BlockSpec((BLK, PROJ_DIM), lambda i: (i, 0)),
            pl.BlockSpec((CLASS_NUM, BLK), lambda i: (0, i)),
            pl.BlockSpec((NQ, PROJ_DIM), lambda i: (0, 0)),
            pl.BlockSpec((NQ, CPAD), lambda i: (0, 0)),
        ],
        out_specs=pl.BlockSpec((1, 1), lambda i: (0, 0), memory_space=pltpu.SMEM),
        out_shape=jax.ShapeDtypeStruct((1, 1), jnp.float32),
    )(activation, ema_activation, pl_t, queue_list, seg)
    return out[0, 0] / batch


# speedup vs baseline: 9.2170x; 1.0461x over previous
import jax
import jax.numpy as jnp
import numpy as np
from jax.experimental import pallas as pl
from jax.experimental.pallas import tpu as pltpu

PROJ_DIM = 128
CLASS_NUM = 7
QUEUE_SIZE = 150
NQ = CLASS_NUM * QUEUE_SIZE
TEMPERATURE = 0.5
INV_T = 1.0 / TEMPERATURE
LN2 = float(np.log(2.0))
C_SCALE = (TEMPERATURE * LN2) ** 2
BLK = 2048
CPAD = 8


def _rowsum_mxu(x):
    ones = jnp.ones((x.shape[1], 1), dtype=jnp.float32)
    return jax.lax.dot_general(
        x, ones, (((1,), (0,)), ((), ())), preferred_element_type=jnp.float32
    )


def _pgc_kernel(act_ref, ema_ref, lab_ref, q_ref, seg_ref, out_ref):
    a = act_ref[...]
    e = ema_ref[...]
    pt = lab_ref[...]
    q = q_ref[...]

    ssa = _rowsum_mxu(a * a)
    sse = _rowsum_mxu(e * e)
    dot_ae = _rowsum_mxu(a * e)

    inv_at = jax.lax.rsqrt(jnp.maximum(ssa * C_SCALE, 1e-24 * C_SCALE))
    inv_e = jax.lax.rsqrt(jnp.maximum(sse, 1e-24))
    spos = dot_ae * inv_at * inv_e

    an_t = a * inv_at
    s = jax.lax.dot_general(
        an_t, q, (((1,), (1,)), ((), ())), preferred_element_type=jnp.float32
    )
    sims = jnp.exp2(s)
    total = jnp.sum(sims, axis=1, keepdims=True)

    colmax = jnp.max(pt, axis=0, keepdims=True)
    cls_idx = jax.lax.broadcasted_iota(jnp.int32, pt.shape, 0)
    label = jnp.min(
        jnp.where(pt == colmax, cls_idx, CLASS_NUM), axis=0, keepdims=True
    )
    onehot_t = (
        jax.lax.broadcasted_iota(jnp.int32, (CPAD, pt.shape[1]), 0) == label
    ).astype(jnp.float32)

    qsum = jax.lax.dot_general(
        seg_ref[...], q, (((0,), (0,)), ((), ())), preferred_element_type=jnp.float32
    )
    g = jax.lax.dot_general(
        onehot_t, an_t, (((1,), (0,)), ((), ())), preferred_element_type=jnp.float32
    )
    pos_total = jnp.sum(qsum * g)

    denom = jnp.exp2(spos) + total
    partial = (
        (QUEUE_SIZE + 1) * jnp.sum(jnp.log2(denom)) - jnp.sum(spos) - pos_total
    ) * (LN2 / (QUEUE_SIZE + 1))

    @pl.when(pl.program_id(0) == 0)
    def _init():
        out_ref[0, 0] = 0.0

    out_ref[0, 0] += partial


def kernel(activation, ema_activation, pseudo_label, queue_list):
    batch = activation.shape[0]
    grid = batch // BLK
    seg_np = np.zeros((NQ, CPAD), dtype=np.float32)
    for c in range(CLASS_NUM):
        seg_np[c * QUEUE_SIZE : (c + 1) * QUEUE_SIZE, c] = 1.0
    seg = jnp.asarray(seg_np)
    pl_t = pseudo_label.T
    out = pl.pallas_call(
        _pgc_kernel,
        grid=(grid,),
        in_specs=[
            pl.BlockSpec((BLK, PROJ_DIM), lambda i: (i, 0)),
            pl.BlockSpec((BLK, PROJ_DIM), lambda i: (i, 0)),
            pl.BlockSpec((CLASS_NUM, BLK), lambda i: (0, i)),
            pl.BlockSpec((NQ, PROJ_DIM), lambda i: (0, 0)),
            pl.BlockSpec((NQ, CPAD), lambda i: (0, 0)),
        ],
        out_specs=pl.BlockSpec((1, 1), lambda i: (0, 0), memory_space=pltpu.SMEM),
        out_shape=jax.ShapeDtypeStruct((1, 1), jnp.float32),
    )(activation, ema_activation, pl_t, queue_list, seg)
    return out[0, 0] / batch


# speedup vs baseline: 9.4425x; 1.0245x over previous
import jax
import jax.numpy as jnp
import numpy as np
from jax.experimental import pallas as pl
from jax.experimental.pallas import tpu as pltpu

PROJ_DIM = 128
CLASS_NUM = 7
QUEUE_SIZE = 150
NQ = CLASS_NUM * QUEUE_SIZE
TEMPERATURE = 0.5
INV_T = 1.0 / TEMPERATURE
LN2 = float(np.log(2.0))
C_SCALE = (TEMPERATURE * LN2) ** 2
BLK = 4096
CPAD = 8


def _rowsum_mxu(x):
    ones = jnp.ones((x.shape[1], 1), dtype=jnp.float32)
    return jax.lax.dot_general(
        x, ones, (((1,), (0,)), ((), ())), preferred_element_type=jnp.float32
    )


def _pgc_kernel(act_ref, ema_ref, lab_ref, q_ref, seg_ref, out_ref):
    a = act_ref[...]
    e = ema_ref[...]
    pt = lab_ref[...]
    q = q_ref[...]

    ssa = _rowsum_mxu(a * a)
    sse = _rowsum_mxu(e * e)
    dot_ae = _rowsum_mxu(a * e)

    inv_at = jax.lax.rsqrt(jnp.maximum(ssa * C_SCALE, 1e-24 * C_SCALE))
    inv_e = jax.lax.rsqrt(jnp.maximum(sse, 1e-24))
    spos = dot_ae * inv_at * inv_e

    an_t = a * inv_at
    s = jax.lax.dot_general(
        an_t, q, (((1,), (1,)), ((), ())), preferred_element_type=jnp.float32
    )
    sims = jnp.exp2(s)
    total = jnp.sum(sims, axis=1, keepdims=True)

    colmax = jnp.max(pt, axis=0, keepdims=True)
    cls_idx = jax.lax.broadcasted_iota(jnp.int32, pt.shape, 0)
    label = jnp.min(
        jnp.where(pt == colmax, cls_idx, CLASS_NUM), axis=0, keepdims=True
    )
    onehot_t = (
        jax.lax.broadcasted_iota(jnp.int32, (CPAD, pt.shape[1]), 0) == label
    ).astype(jnp.float32)

    qsum = jax.lax.dot_general(
        seg_ref[...], q, (((0,), (0,)), ((), ())), preferred_element_type=jnp.float32
    )
    g = jax.lax.dot_general(
        onehot_t, an_t, (((1,), (0,)), ((), ())), preferred_element_type=jnp.float32
    )
    pos_total = jnp.sum(qsum * g)

    denom = jnp.exp2(spos) + total
    partial = (
        (QUEUE_SIZE + 1) * jnp.sum(jnp.log2(denom)) - jnp.sum(spos) - pos_total
    ) * (LN2 / (QUEUE_SIZE + 1))

    @pl.when(pl.program_id(0) == 0)
    def _init():
        out_ref[0, 0] = 0.0

    out_ref[0, 0] += partial


def kernel(activation, ema_activation, pseudo_label, queue_list):
    batch = activation.shape[0]
    grid = batch // BLK
    seg_np = np.zeros((NQ, CPAD), dtype=np.float32)
    for c in range(CLASS_NUM):
        seg_np[c * QUEUE_SIZE : (c + 1) * QUEUE_SIZE, c] = 1.0
    seg = jnp.asarray(seg_np)
    pl_t = pseudo_label.T
    out = pl.pallas_call(
        _pgc_kernel,
        grid=(grid,),
        in_specs=[
            pl.BlockSpec((BLK, PROJ_DIM), lambda i: (i, 0)),
            pl.BlockSpec((BLK, PROJ_DIM), lambda i: (i, 0)),
            pl.BlockSpec((CLASS_NUM, BLK), lambda i: (0, i)),
            pl.BlockSpec((NQ, PROJ_DIM), lambda i: (0, 0)),
            pl.BlockSpec((NQ, CPAD), lambda i: (0, 0)),
        ],
        out_specs=pl.BlockSpec((1, 1), lambda i: (0, 0), memory_space=pltpu.SMEM),
        out_shape=jax.ShapeDtypeStruct((1, 1), jnp.float32),
    )(activation, ema_activation, pl_t, queue_list, seg)
    return out[0, 0] / batch
